# Initial kernel scaffold; baseline (speedup 1.0000x reference)
import jax
import jax.numpy as jnp
from jax.experimental import pallas as pl


def kernel(x, W):
    raise NotImplementedError("write your pallas kernel here")


# speedup vs baseline: 1.5495x; 1.5495x over previous
import functools

import jax
import jax.numpy as jnp
from jax.experimental import pallas as pl

_DIM = 768
_N_EXPERTS = 8
_TOP_K = 2
_BLOCK = 2048


def _router_body(x_ref, w_ref, c_ref, idx_ref):
    x = x_ref[...]
    w = w_ref[...]
    scores = jax.lax.dot_general(
        x, w, (((1,), (1,)), ((), ())), preferred_element_type=jnp.float32
    )
    e_iota = jax.lax.broadcasted_iota(jnp.int32, scores.shape, 1)

    m1 = jnp.max(scores, axis=-1, keepdims=True)
    i1 = jnp.min(
        jnp.where(scores == m1, e_iota, _N_EXPERTS), axis=-1, keepdims=True
    )
    masked = jnp.where(e_iota == i1, -jnp.inf, scores)
    m2 = jnp.max(masked, axis=-1, keepdims=True)
    i2 = jnp.min(
        jnp.where(masked == m2, e_iota, _N_EXPERTS), axis=-1, keepdims=True
    )

    z = jnp.sum(jnp.exp(scores - m1), axis=-1, keepdims=True)
    c1 = 1.0 / z
    c2 = jnp.exp(m2 - m1) / z

    c_ref[...] = jnp.concatenate([c1, c2], axis=-1)
    idx_ref[...] = jnp.concatenate([i1, i2], axis=-1).astype(jnp.int32)


@jax.jit
def kernel(x, W):
    tokens = x.shape[0]
    grid = (tokens // _BLOCK,)
    c, idx = pl.pallas_call(
        _router_body,
        grid=grid,
        in_specs=[
            pl.BlockSpec((_BLOCK, _DIM), lambda i: (i, 0)),
            pl.BlockSpec((_N_EXPERTS, _DIM), lambda i: (0, 0)),
        ],
        out_specs=[
            pl.BlockSpec((_BLOCK, _TOP_K), lambda i: (i, 0)),
            pl.BlockSpec((_BLOCK, _TOP_K), lambda i: (i, 0)),
        ],
        out_shape=[
            jax.ShapeDtypeStruct((tokens, _TOP_K), jnp.float32),
            jax.ShapeDtypeStruct((tokens, _TOP_K), jnp.int32),
        ],
    )(x, W)
    return (c, idx)


# speedup vs baseline: 3.3534x; 2.1642x over previous
import jax
import jax.numpy as jnp
from jax.experimental import pallas as pl

_DIM = 768
_N_EXPERTS = 8
_TOP_K = 2
_BLOCK = 2048


def _router_body(x_ref, w_ref, c_ref, idx_ref):
    x = x_ref[...]
    w = w_ref[...]
    scores = jax.lax.dot_general(
        w, x, (((1,), (1,)), ((), ())), preferred_element_type=jnp.float32
    )
    e_iota = jax.lax.broadcasted_iota(jnp.int32, scores.shape, 0)

    m1 = jnp.max(scores, axis=0, keepdims=True)
    i1 = jnp.min(
        jnp.where(scores == m1, e_iota, _N_EXPERTS), axis=0, keepdims=True
    )
    masked = jnp.where(e_iota == i1, -jnp.inf, scores)
    m2 = jnp.max(masked, axis=0, keepdims=True)
    i2 = jnp.min(
        jnp.where(masked == m2, e_iota, _N_EXPERTS), axis=0, keepdims=True
    )

    z = jnp.sum(jnp.exp(scores - m1), axis=0, keepdims=True)
    c1 = 1.0 / z
    c2 = jnp.exp(m2 - m1) / z

    c_ref[...] = jnp.concatenate([c1, c2], axis=0)
    idx_ref[...] = jnp.concatenate([i1, i2], axis=0).astype(jnp.int32)


@jax.jit
def kernel(x, W):
    tokens = x.shape[0]
    grid = (tokens // _BLOCK,)
    c_t, idx_t = pl.pallas_call(
        _router_body,
        grid=grid,
        in_specs=[
            pl.BlockSpec((_BLOCK, _DIM), lambda i: (i, 0)),
            pl.BlockSpec((_N_EXPERTS, _DIM), lambda i: (0, 0)),
        ],
        out_specs=[
            pl.BlockSpec((_TOP_K, _BLOCK), lambda i: (0, i)),
            pl.BlockSpec((_TOP_K, _BLOCK), lambda i: (0, i)),
        ],
        out_shape=[
            jax.ShapeDtypeStruct((_TOP_K, tokens), jnp.float32),
            jax.ShapeDtypeStruct((_TOP_K, tokens), jnp.int32),
        ],
    )(x, W)
    return (c_t.T, idx_t.T)


# speedup vs baseline: 3.4620x; 1.0324x over previous
import jax
import jax.numpy as jnp
from jax.experimental import pallas as pl

_DIM = 768
_N_EXPERTS = 8
_TOP_K = 2
_BLOCK = 4096


def _router_body(x_ref, w_ref, c_ref, idx_ref):
    x = x_ref[...]
    w = w_ref[...]
    scores = jax.lax.dot_general(
        w, x, (((1,), (1,)), ((), ())), preferred_element_type=jnp.float32
    )
    e_iota = jax.lax.broadcasted_iota(jnp.int32, scores.shape, 0)

    m1 = jnp.max(scores, axis=0, keepdims=True)
    i1 = jnp.min(
        jnp.where(scores == m1, e_iota, _N_EXPERTS), axis=0, keepdims=True
    )
    masked = jnp.where(e_iota == i1, -jnp.inf, scores)
    m2 = jnp.max(masked, axis=0, keepdims=True)
    i2 = jnp.min(
        jnp.where(masked == m2, e_iota, _N_EXPERTS), axis=0, keepdims=True
    )

    z = jnp.sum(jnp.exp(scores - m1), axis=0, keepdims=True)
    c1 = 1.0 / z
    c2 = jnp.exp(m2 - m1) / z

    c_ref[...] = jnp.concatenate([c1, c2], axis=0)
    idx_ref[...] = jnp.concatenate([i1, i2], axis=0).astype(jnp.int32)


@jax.jit
def kernel(x, W):
    tokens = x.shape[0]
    grid = (tokens // _BLOCK,)
    c_t, idx_t = pl.pallas_call(
        _router_body,
        grid=grid,
        in_specs=[
            pl.BlockSpec((_BLOCK, _DIM), lambda i: (i, 0)),
            pl.BlockSpec((_N_EXPERTS, _DIM), lambda i: (0, 0)),
        ],
        out_specs=[
            pl.BlockSpec((_TOP_K, _BLOCK), lambda i: (0, i)),
            pl.BlockSpec((_TOP_K, _BLOCK), lambda i: (0, i)),
        ],
        out_shape=[
            jax.ShapeDtypeStruct((_TOP_K, tokens), jnp.float32),
            jax.ShapeDtypeStruct((_TOP_K, tokens), jnp.int32),
        ],
    )(x, W)
    return (c_t.T, idx_t.T)
